# baseline (speedup 1.0000x reference)
import jax
import jax.numpy as jnp
from jax.experimental import pallas as pl
from jax.experimental.pallas import tpu as pltpu
from jax.experimental.pallas import tpu_sc as plsc

B = 64
D_OBS = 768
MOD = 32
HID = 2048
L = 64
CD = 32
K = 8192
BL = B * L

KBLK = 512
NBLK = K // KBLK


def _enc_body(obs_ref, act_ref, w1_ref, b1_ref, w2_ref, b2_ref, z_ref):
    x = jnp.concatenate([obs_ref[...], act_ref[...]], axis=1)
    h = jnp.maximum(jnp.dot(x, w1_ref[...]) + b1_ref[...], 0.0)
    z_ref[...] = jnp.dot(h, w2_ref[...]) + b2_ref[...]


def _vq_body(zrt_ref, cb3_ref, codes_ref):
    zrt = zrt_ref[...]
    ze_sq = jnp.sum(zrt * zrt, axis=0, keepdims=True)

    ii8 = jax.lax.broadcasted_iota(jnp.int32, (8, 1), 0).astype(jnp.float32)

    def dots_blk(blk):
        cb = cb3_ref[pl.ds(blk * KBLK, KBLK), :] * -2.0
        cb_sq = 0.25 * jnp.sum(cb * cb, axis=1, keepdims=True)
        dots2 = jax.lax.dot_general(
            cb, zrt, dimension_numbers=(((1,), (0,)), ((), ())))
        return dots2, cb_sq

    def argmin_blk(blk, dots2, cb_sq, best_d, best_i):
        cur_d = (ze_sq + dots2[0:8, :]) + cb_sq[0:8, :]
        cur_i = jnp.broadcast_to(ii8, (8, BL))
        for r in range(1, KBLK // 8):
            d2r = (ze_sq + dots2[r * 8:(r + 1) * 8, :]) + cb_sq[r * 8:(r + 1) * 8, :]
            takeb = d2r < cur_d
            cur_d = jnp.minimum(cur_d, d2r)
            cur_i = jnp.where(takeb, ii8 + float(r * 8), cur_i)
        for h in (4, 2, 1):
            da, db = cur_d[:h, :], cur_d[h:2 * h, :]
            ia, ib = cur_i[:h, :], cur_i[h:2 * h, :]
            takeb = (db < da) | ((db == da) & (ib < ia))
            cur_d = jnp.where(takeb, db, da)
            cur_i = jnp.where(takeb, ib, ia)
        m = cur_d
        li = cur_i + (blk * KBLK).astype(jnp.float32)
        upd = m < best_d
        return (jnp.where(upd, m, best_d), jnp.where(upd, li, best_i))

    def step(i2, carry):
        best_d, best_i = carry
        b0 = i2 * 2
        b1 = b0 + 1
        dots0, csq0 = dots_blk(b0)
        dots1, csq1 = dots_blk(b1)
        best_d, best_i = argmin_blk(b0, dots0, csq0, best_d, best_i)
        best_d, best_i = argmin_blk(b1, dots1, csq1, best_d, best_i)
        return (best_d, best_i)

    best_d = jnp.full((1, BL), jnp.inf, jnp.float32)
    best_i = jnp.zeros((1, BL), jnp.float32)
    _, best_i = jax.lax.fori_loop(0, NBLK // 2, step, (best_d, best_i))
    codes_ref[...] = best_i.astype(jnp.int32)


def _dec_body(ze_ref, zq_ref, w1_ref, b1_ref, w2_ref, b2_ref,
              recon_ref, loss_ref):
    ze = ze_ref[...]
    zq = zq_ref[...]
    diff = ze - zq
    l0 = jnp.sum(diff * diff) / float(BL * CD)
    loss_ref[...] = jnp.reshape(l0 + 0.25 * l0, (1, 1))
    zst = ze + (zq - ze)
    h2 = jnp.maximum(jnp.dot(zst, w1_ref[...]) + b1_ref[...], 0.0)
    recon_ref[...] = jnp.dot(h2, w2_ref[...]) + b2_ref[...]


def _sc_gather(cb_pad, codes_row):
    window = 128
    mesh = plsc.VectorSubcoreMesh(core_axis_name="c", subcore_axis_name="s")

    @pl.kernel(out_type=jax.ShapeDtypeStruct((BL, 128), cb_pad.dtype),
               mesh=mesh)
    def kern(cb_hbm, i_hbm, o_hbm):
        def body(i_vmem, o_vmem):
            pltpu.sync_copy(cb_hbm.at[i_vmem.at[0]], o_vmem)

        pltpu.emit_pipeline(
            body,
            grid=(BL // window,),
            in_specs=[pl.BlockSpec((1, window), index_map=lambda i: (0, i))],
            out_specs=[pl.BlockSpec((window, 128), index_map=lambda i: (i, 0))],
            core_axis_name=("c", "s"),
            dimension_semantics=(pltpu.PARALLEL,),
        )(i_hbm, o_hbm)

    return kern(cb_pad, codes_row)


def kernel(obs, action, W_enc1, b_enc1, W_enc2, b_enc2, codebook,
           W_dec1, b_dec1, W_dec2, b_dec2):
    f32 = jnp.float32
    z_e = pl.pallas_call(
        _enc_body,
        out_shape=jax.ShapeDtypeStruct((B, HID), f32),
    )(obs, action, W_enc1, b_enc1.reshape(1, HID), W_enc2,
      b_enc2.reshape(1, L * CD))

    zr = z_e.reshape(BL, CD)
    codes = pl.pallas_call(
        _vq_body,
        out_shape=jax.ShapeDtypeStruct((1, BL), jnp.int32),
    )(zr.T, codebook)

    cb_pad = jnp.pad(codebook, ((0, 0), (0, 128 - CD)))
    z_q = _sc_gather(cb_pad, codes)[:, :CD]

    recon, loss = pl.pallas_call(
        _dec_body,
        out_shape=[jax.ShapeDtypeStruct((B, D_OBS), f32),
                   jax.ShapeDtypeStruct((1, 1), f32)],
    )(z_e, z_q.reshape(B, L * CD), W_dec1, b_dec1.reshape(1, HID),
      W_dec2, b_dec2.reshape(1, D_OBS))

    return recon, loss.reshape(())


# speedup vs baseline: 1.0102x; 1.0102x over previous
import jax
import jax.numpy as jnp
from jax.experimental import pallas as pl
from jax.experimental.pallas import tpu as pltpu
from jax.experimental.pallas import tpu_sc as plsc

B = 64
D_OBS = 768
MOD = 32
HID = 2048
L = 64
CD = 32
K = 8192
BL = B * L

KBLK = 512
NBLK = K // KBLK


def _enc_body(obs_ref, act_ref, w1_ref, b1_ref, w2_ref, b2_ref, z_ref):
    x = jnp.concatenate([obs_ref[...], act_ref[...]], axis=1)
    h = jnp.maximum(jnp.dot(x, w1_ref[...]) + b1_ref[...], 0.0)
    z_ref[...] = jnp.dot(h, w2_ref[...]) + b2_ref[...]


def _vq_body(zrt_ref, cb3_ref, codes_ref):
    zrt = zrt_ref[...]
    ze_sq = jnp.sum(zrt * zrt, axis=0, keepdims=True)

    ii8 = jax.lax.broadcasted_iota(jnp.int32, (8, 1), 0).astype(jnp.float32)

    def dots_blk(blk):
        cb = cb3_ref[blk] * -2.0
        cb_sq = 0.25 * jnp.sum(cb * cb, axis=1, keepdims=True)
        dots2 = jax.lax.dot_general(
            cb, zrt, dimension_numbers=(((1,), (0,)), ((), ())))
        return dots2, cb_sq

    def argmin_blk(blk, dots2, cb_sq, best_d, best_i):
        cur_d = (ze_sq + dots2[0:8, :]) + cb_sq[0:8, :]
        cur_i = jnp.broadcast_to(ii8, (8, BL))
        for r in range(1, KBLK // 8):
            d2r = (ze_sq + dots2[r * 8:(r + 1) * 8, :]) + cb_sq[r * 8:(r + 1) * 8, :]
            takeb = d2r < cur_d
            cur_d = jnp.minimum(cur_d, d2r)
            cur_i = jnp.where(takeb, ii8 + float(r * 8), cur_i)
        for h in (4, 2, 1):
            da, db = cur_d[:h, :], cur_d[h:2 * h, :]
            ia, ib = cur_i[:h, :], cur_i[h:2 * h, :]
            takeb = (db < da) | ((db == da) & (ib < ia))
            cur_d = jnp.where(takeb, db, da)
            cur_i = jnp.where(takeb, ib, ia)
        m = cur_d
        li = cur_i + (blk * KBLK).astype(jnp.float32)
        upd = m < best_d
        return (jnp.where(upd, m, best_d), jnp.where(upd, li, best_i))

    def step(i2, carry):
        best_d, best_i = carry
        b0 = i2 * 2
        b1 = b0 + 1
        dots0, csq0 = dots_blk(b0)
        dots1, csq1 = dots_blk(b1)
        best_d, best_i = argmin_blk(b0, dots0, csq0, best_d, best_i)
        best_d, best_i = argmin_blk(b1, dots1, csq1, best_d, best_i)
        return (best_d, best_i)

    best_d = jnp.full((1, BL), jnp.inf, jnp.float32)
    best_i = jnp.zeros((1, BL), jnp.float32)
    _, best_i = jax.lax.fori_loop(0, NBLK // 2, step, (best_d, best_i))
    codes_ref[...] = best_i.astype(jnp.int32)


def _dec_body(ze_ref, zq_ref, w1_ref, b1_ref, w2_ref, b2_ref,
              recon_ref, loss_ref):
    ze = ze_ref[...]
    zq = zq_ref[...]
    diff = ze - zq
    l0 = jnp.sum(diff * diff) / float(BL * CD)
    loss_ref[...] = jnp.reshape(l0 + 0.25 * l0, (1, 1))
    zst = ze + (zq - ze)
    h2 = jnp.maximum(jnp.dot(zst, w1_ref[...]) + b1_ref[...], 0.0)
    recon_ref[...] = jnp.dot(h2, w2_ref[...]) + b2_ref[...]


def _sc_gather(cb_pad, codes_row):
    window = 128
    mesh = plsc.VectorSubcoreMesh(core_axis_name="c", subcore_axis_name="s")

    @pl.kernel(out_type=jax.ShapeDtypeStruct((BL, 128), cb_pad.dtype),
               mesh=mesh)
    def kern(cb_hbm, i_hbm, o_hbm):
        def body(i_vmem, o_vmem):
            pltpu.sync_copy(cb_hbm.at[i_vmem.at[0]], o_vmem)

        pltpu.emit_pipeline(
            body,
            grid=(BL // window,),
            in_specs=[pl.BlockSpec((1, window), index_map=lambda i: (0, i))],
            out_specs=[pl.BlockSpec((window, 128), index_map=lambda i: (i, 0))],
            core_axis_name=("c", "s"),
            dimension_semantics=(pltpu.PARALLEL,),
        )(i_hbm, o_hbm)

    return kern(cb_pad, codes_row)


def kernel(obs, action, W_enc1, b_enc1, W_enc2, b_enc2, codebook,
           W_dec1, b_dec1, W_dec2, b_dec2):
    f32 = jnp.float32
    z_e = pl.pallas_call(
        _enc_body,
        out_shape=jax.ShapeDtypeStruct((B, HID), f32),
    )(obs, action, W_enc1, b_enc1.reshape(1, HID), W_enc2,
      b_enc2.reshape(1, L * CD))

    zr = z_e.reshape(BL, CD)
    cb3 = codebook.reshape(NBLK, KBLK, CD)
    codes = pl.pallas_call(
        _vq_body,
        out_shape=jax.ShapeDtypeStruct((1, BL), jnp.int32),
    )(zr.T, cb3)

    cb_pad = jnp.pad(codebook, ((0, 0), (0, 128 - CD)))
    z_q = _sc_gather(cb_pad, codes)[:, :CD]

    recon, loss = pl.pallas_call(
        _dec_body,
        out_shape=[jax.ShapeDtypeStruct((B, D_OBS), f32),
                   jax.ShapeDtypeStruct((1, 1), f32)],
    )(z_e, z_q.reshape(B, L * CD), W_dec1, b_dec1.reshape(1, HID),
      W_dec2, b_dec2.reshape(1, D_OBS))

    return recon, loss.reshape(())
